# baseline (speedup 1.0000x reference)
import jax
import jax.numpy as jnp
from jax.experimental import pallas as pl
from jax.experimental.pallas import tpu as pltpu

_BB = 256
_PB = 8


def _project(x_ref, w_ref, out_scratch):
    t = jax.lax.dot_general(
        x_ref[...], w_ref[...],
        dimension_numbers=(((1,), (1,)), ((), ())),
        preferred_element_type=jnp.float32,
    )
    mu = jnp.mean(t, axis=1, keepdims=True)
    var = jnp.mean(jnp.square(t - mu), axis=1, keepdims=True)
    out_scratch[...] = t * jax.lax.rsqrt(var + 1e-5)


def _sweep(t_scratch, cbts, out_ref):
    tts = t_scratch[...]
    for i in range(_BB // _PB):
        chunk = tts[i * _PB:(i + 1) * _PB, :]
        d = chunk[:, :, None] - cbts[None, :, :]
        out_ref[i * _PB:(i + 1) * _PB, :] = jnp.argmin(d, axis=1).astype(jnp.int32)


def _rpq_kernel(x_ref, w_ref, cbt_ref, out_ref, s0, s1):
    g = pl.program_id(0)
    n = pl.num_programs(0) - 1
    cbt = cbt_ref[...]
    cmu = jnp.mean(cbt)
    cvar = jnp.mean(jnp.square(cbt - cmu))
    cbts = cbt * jax.lax.rsqrt(cvar + 1e-5)

    even = (g % 2) == 0

    @pl.when(jnp.logical_and(even, g == 0))
    def _():
        _project(x_ref, w_ref, s0)

    @pl.when(jnp.logical_and(g > 0, jnp.logical_not(even)))
    def _():
        _project(x_ref, w_ref, s1)
        _sweep(s0, cbts, out_ref)

    @pl.when(jnp.logical_and(g > 0, even))
    def _():
        _project(x_ref, w_ref, s0)
        _sweep(s1, cbts, out_ref)


def kernel(input_values, W, code_book, raw_signal):
    B, D = input_values.shape
    Q = W.shape[0]
    n = B // _BB
    cbt = code_book[:Q].T
    return pl.pallas_call(
        _rpq_kernel,
        grid=(n + 1,),
        in_specs=[
            pl.BlockSpec((_BB, D), lambda g: (jnp.minimum(g, n - 1), 0)),
            pl.BlockSpec((Q, D), lambda g: (0, 0)),
            pl.BlockSpec((Q, Q), lambda g: (0, 0)),
        ],
        out_specs=pl.BlockSpec((_BB, Q), lambda g: (jnp.maximum(g - 1, 0), 0)),
        out_shape=jax.ShapeDtypeStruct((B, Q), jnp.int32),
        scratch_shapes=[
            pltpu.VMEM((_BB, Q), jnp.float32),
            pltpu.VMEM((_BB, Q), jnp.float32),
        ],
    )(input_values, W, cbt)


# speedup vs baseline: 1.0643x; 1.0643x over previous
import jax
import jax.numpy as jnp
from jax.experimental import pallas as pl

_BB = 256
_PB = 8


def _rpq_kernel(x_ref, w_ref, cbt_ref, out_ref):
    t = jax.lax.dot_general(
        x_ref[...], w_ref[...],
        dimension_numbers=(((1,), (1,)), ((), ())),
        preferred_element_type=jnp.float32,
    )
    mu = jnp.mean(t, axis=1, keepdims=True)
    var = jnp.mean(jnp.square(t - mu), axis=1, keepdims=True)
    tts = t * jax.lax.rsqrt(var + 1e-5)
    cbt = cbt_ref[...]
    cmu = jnp.mean(cbt)
    cvar = jnp.mean(jnp.square(cbt - cmu))
    cbts = cbt * jax.lax.rsqrt(cvar + 1e-5)

    for i in range(_BB // _PB):
        chunk = tts[i * _PB:(i + 1) * _PB, :]
        d = chunk[:, :, None] - cbts[None, :, :]
        out_ref[i * _PB:(i + 1) * _PB, :] = jnp.argmin(d, axis=1).astype(jnp.int32)


def kernel(input_values, W, code_book, raw_signal):
    B, D = input_values.shape
    Q = W.shape[0]
    cbt = code_book[:Q].T
    return pl.pallas_call(
        _rpq_kernel,
        grid=(B // _BB,),
        in_specs=[
            pl.BlockSpec((_BB, D), lambda i: (i, 0)),
            pl.BlockSpec((Q, D), lambda i: (0, 0)),
            pl.BlockSpec((Q, Q), lambda i: (0, 0)),
        ],
        out_specs=pl.BlockSpec((_BB, Q), lambda i: (i, 0)),
        out_shape=jax.ShapeDtypeStruct((B, Q), jnp.int32),
    )(input_values, W, cbt)
